# Initial kernel scaffold; baseline (speedup 1.0000x reference)
import jax
import jax.numpy as jnp
from jax.experimental import pallas as pl


def kernel(authdir, year, age, actor, rated, genre, occu, W_authdir, W_year, W_age, W_actor, W_rated, W_genre, W_occu):
    raise NotImplementedError("write your pallas kernel here")


# speedup vs baseline: 1.2709x; 1.2709x over previous
import jax
import jax.numpy as jnp
from jax import lax
from jax.experimental import pallas as pl
from jax.experimental.pallas import tpu as pltpu
from jax.experimental.pallas import tpu_sc as plsc

_B = 16384
_EMB = 32
_NT = 7
_OUT_D = _NT * _EMB

_NC = 2
_NS = 16
_NW = _NC * _NS
_BPW = _B // _NW
_CHUNK = 128
_NCH = _BPW // _CHUNK


def _body(*refs):
    idx_hbm = refs[0:_NT]
    tbl_hbm = refs[_NT:2 * _NT]
    out_hbm = refs[2 * _NT]
    idx_v = refs[2 * _NT + 1:3 * _NT + 1]
    rows_v = refs[3 * _NT + 1:4 * _NT + 1]
    sem = refs[4 * _NT + 1]

    wid = lax.axis_index("s") * _NC + lax.axis_index("c")
    base = wid * _BPW

    handles = []
    for t in range(_NT):
        for j in range(_NCH):
            pltpu.sync_copy(
                idx_hbm[t].at[pl.ds(base + j * _CHUNK, _CHUNK)],
                idx_v[t].at[j])
        for j in range(_NCH):
            handles.append(
                pltpu.async_copy(
                    tbl_hbm[t].at[idx_v[t].at[j]],
                    rows_v[t].at[pl.ds(j * _CHUNK, _CHUNK)],
                    sem))
    for h in handles:
        h.wait()
    for t in range(_NT):
        pltpu.sync_copy(
            rows_v[t],
            out_hbm.at[pl.ds(base, _BPW), pl.ds(t * _EMB, _EMB)])


@jax.jit
def kernel(authdir, year, age, actor, rated, genre, occu,
           W_authdir, W_year, W_age, W_actor, W_rated, W_genre, W_occu):
    mesh = plsc.VectorSubcoreMesh(core_axis_name="c", subcore_axis_name="s")
    scratch = (
        [pltpu.VMEM((_NCH, _CHUNK), jnp.int32) for _ in range(_NT)]
        + [pltpu.VMEM((_BPW, _EMB), jnp.float32) for _ in range(_NT)]
        + [pltpu.SemaphoreType.DMA])
    f = pl.kernel(
        _body,
        out_type=jax.ShapeDtypeStruct((_B, _OUT_D), jnp.float32),
        mesh=mesh,
        scratch_types=scratch,
        compiler_params=pltpu.CompilerParams(use_tc_tiling_on_sc=False))
    return f(authdir, year, age, actor, rated, genre, occu,
             W_authdir, W_year, W_age, W_actor, W_rated, W_genre, W_occu)


# speedup vs baseline: 1.3454x; 1.0586x over previous
import jax
import jax.numpy as jnp
from jax import lax
from jax.experimental import pallas as pl
from jax.experimental.pallas import tpu as pltpu
from jax.experimental.pallas import tpu_sc as plsc

_B = 16384
_EMB = 32
_NT = 7
_OUT_D = _NT * _EMB

_NC = 2
_NS = 16
_NW = _NC * _NS
_BPW = _B // _NW
_CHUNK = 128
_NCH = _BPW // _CHUNK


def _body(*refs):
    idx_hbm = refs[0:_NT]
    tbl_hbm = refs[_NT:2 * _NT]
    out_hbm = refs[2 * _NT]
    idx_v = refs[2 * _NT + 1:3 * _NT + 1]
    rows_v = refs[3 * _NT + 1:4 * _NT + 1]
    sem_i = refs[4 * _NT + 1]
    sem_g = refs[4 * _NT + 2]
    sem_o = refs[4 * _NT + 3]

    wid = lax.axis_index("s") * _NC + lax.axis_index("c")
    base = wid * _BPW

    ih = []
    for t in range(_NT):
        for j in range(_NCH):
            ih.append(pltpu.async_copy(
                idx_hbm[t].at[pl.ds(base + j * _CHUNK, _CHUNK)],
                idx_v[t].at[j], sem_i.at[t]))
    gh = []
    for t in range(_NT):
        for j in range(_NCH):
            ih[t * _NCH + j].wait()
        for j in range(_NCH):
            gh.append(pltpu.async_copy(
                tbl_hbm[t].at[idx_v[t].at[j]],
                rows_v[t].at[pl.ds(j * _CHUNK, _CHUNK)],
                sem_g.at[t]))
    oh = []
    for t in range(_NT):
        for j in range(_NCH):
            gh[t * _NCH + j].wait()
        oh.append(pltpu.async_copy(
            rows_v[t],
            out_hbm.at[pl.ds(base, _BPW), pl.ds(t * _EMB, _EMB)],
            sem_o))
    for h in oh:
        h.wait()


@jax.jit
def kernel(authdir, year, age, actor, rated, genre, occu,
           W_authdir, W_year, W_age, W_actor, W_rated, W_genre, W_occu):
    mesh = plsc.VectorSubcoreMesh(core_axis_name="c", subcore_axis_name="s")
    scratch = (
        [pltpu.VMEM((_NCH, _CHUNK), jnp.int32) for _ in range(_NT)]
        + [pltpu.VMEM((_BPW, _EMB), jnp.float32) for _ in range(_NT)]
        + [pltpu.SemaphoreType.DMA((_NT,)),
           pltpu.SemaphoreType.DMA((_NT,)),
           pltpu.SemaphoreType.DMA])
    f = pl.kernel(
        _body,
        out_type=jax.ShapeDtypeStruct((_B, _OUT_D), jnp.float32),
        mesh=mesh,
        scratch_types=scratch,
        compiler_params=pltpu.CompilerParams(use_tc_tiling_on_sc=False))
    return f(authdir, year, age, actor, rated, genre, occu,
             W_authdir, W_year, W_age, W_actor, W_rated, W_genre, W_occu)
